# baseline (speedup 1.0000x reference)
import dataclasses
import functools

import jax
import jax.numpy as jnp
from jax import lax
from jax.experimental import pallas as pl
from jax.experimental.pallas import tpu as pltpu
from jax.experimental.pallas import tpu_sc as plsc

N = 10000
E = 320000
F_IN = 128
F_OUT = 64
NP = 10240
BLK = 6400
NBLK = E // BLK
LANES = 16


def _matmul_body(w_ref, x_ref, o_ref):
    o_ref[...] = lax.dot_general(
        w_ref[...], x_ref[...],
        (((1,), (1,)), ((), ())),
        preferred_element_type=jnp.float32,
        precision=lax.Precision.HIGHEST,
    )


def _pack_body(r_ref, c_ref, o_ref):
    o_ref[...] = r_ref[...] | (c_ref[...] << 14)


def _pack_edges(row, col):
    r2 = row.reshape(E // 128, 128)
    c2 = col.reshape(E // 128, 128)
    packed = pl.pallas_call(
        _pack_body,
        out_shape=jax.ShapeDtypeStruct((E // 128, 128), jnp.int32),
    )(r2, c2)
    return packed.reshape(E)


def _feature_major_xw(x_pad, W):
    bn = 2048
    return pl.pallas_call(
        _matmul_body,
        grid=(NP // bn,),
        in_specs=[
            pl.BlockSpec((F_OUT, F_IN), lambda i: (0, 0)),
            pl.BlockSpec((bn, F_IN), lambda i: (i, 0)),
        ],
        out_specs=pl.BlockSpec((F_OUT, bn), lambda i: (0, i)),
        out_shape=jax.ShapeDtypeStruct((F_OUT, NP), jnp.float32),
    )(W, x_pad)


def _rsqrt16(d):
    i = lax.bitcast_convert_type(d, jnp.int32)
    y = lax.bitcast_convert_type(jnp.int32(0x5F3759DF) - (i >> 1), jnp.float32)
    half = d * 0.5
    for _ in range(3):
        y = y * (1.5 - half * y * y)
    return y


def _sc_propagate(xwT, packed, ew):
    mesh = plsc.VectorSubcoreMesh(
        core_axis_name="c", subcore_axis_name="s", num_cores=2, num_subcores=16
    )
    cp = pltpu.CompilerParams()
    if "needs_layout_passes" in pltpu.CompilerParams.__dataclass_fields__:
        cp = dataclasses.replace(cp, needs_layout_passes=False)

    @functools.partial(
        pl.kernel,
        compiler_params=cp,
        out_type=(
            jax.ShapeDtypeStruct((F_OUT, NP), jnp.float32),
            jax.ShapeDtypeStruct((32, NP), jnp.float32),
            jax.ShapeDtypeStruct((2, NP), jnp.float32),
        ),
        mesh=mesh,
        scratch_types=[
            pltpu.VMEM((NP,), jnp.float32),
            pltpu.VMEM((NP,), jnp.float32),
            pltpu.VMEM((NP,), jnp.float32),
            pltpu.VMEM((NP,), jnp.float32),
            pltpu.VMEM((NP,), jnp.float32),
            pltpu.VMEM((NP,), jnp.float32),
            pltpu.VMEM((BLK,), jnp.int32),
            pltpu.VMEM((BLK,), jnp.float32),
            pltpu.VMEM((BLK,), jnp.int32),
            pltpu.VMEM((BLK,), jnp.float32),
            pltpu.VMEM((4000,), jnp.int32),
            pltpu.VMEM((4000,), jnp.float32),
            pltpu.VMEM((NP // 16,), jnp.float32),
            pltpu.SemaphoreType.DMA,
            pltpu.SemaphoreType.DMA,
            pltpu.SemaphoreType.DMA,
        ],
    )
    def sc_prop(xw_hbm, pk_hbm, ew_hbm, out_hbm, part_hbm, degsc_hbm,
                h0, h1, t0, t1, dg, ds,
                pb0, wb0, pb1, wb1, db, dwb, tmp, semA, semB, tsem):
        sid = lax.axis_index("s")
        cid = lax.axis_index("c")
        wid = sid * 2 + cid
        j0 = wid * 2

        tcps = []
        for q in range(4):
            qs = pl.ds(2560 * q, 2560)
            tcps.append(pltpu.async_copy(xw_hbm.at[j0].at[qs], h0.at[qs], tsem))
            tcps.append(
                pltpu.async_copy(xw_hbm.at[j0 + 1].at[qs], h1.at[qs], tsem))

        def issue(e0, p_, w_, sem):
            sl = pl.ds(e0, BLK)
            pltpu.async_copy(pk_hbm.at[sl], p_, sem)
            pltpu.async_copy(ew_hbm.at[sl], w_, sem)

        def drain(p_, w_, sem):
            sl = pl.ds(0, BLK)
            pltpu.make_async_copy(pk_hbm.at[sl], p_, sem).wait()
            pltpu.make_async_copy(ew_hbm.at[sl], w_, sem).wait()

        def edge_pass(chunk_body):
            issue(0, pb0, wb0, semA)

            @pl.loop(0, E, step=2 * BLK)
            def _(e0):
                issue(e0 + BLK, pb1, wb1, semB)
                drain(pb0, wb0, semA)

                @plsc.parallel_loop(0, BLK, step=LANES, unroll=8)
                def _(k):
                    chunk_body(pb0, wb0, k)

                @pl.when(e0 + 2 * BLK < E)
                def _():
                    issue(e0 + 2 * BLK, pb0, wb0, semA)

                drain(pb1, wb1, semB)

                @plsc.parallel_loop(0, BLK, step=LANES, unroll=8)
                def _(k):
                    chunk_body(pb1, wb1, k)

        init16 = jnp.full((LANES,), 1.0, jnp.float32) * (sid == 0).astype(
            jnp.float32)

        @plsc.parallel_loop(0, NP, step=LANES)
        def _(i):
            dg[pl.ds(i, LANES)] = init16

        dbase = sid * (E // 16)
        for rblk in range(5):
            off = dbase + rblk * 4000
            pltpu.async_copy(pk_hbm.at[pl.ds(off, 4000)], db, semA)
            pltpu.async_copy(ew_hbm.at[pl.ds(off, 4000)], dwb, semA)
            pltpu.make_async_copy(pk_hbm.at[pl.ds(0, 4000)], db, semA).wait()
            pltpu.make_async_copy(ew_hbm.at[pl.ds(0, 4000)], dwb, semA).wait()

            @plsc.parallel_loop(0, 4000, step=LANES, unroll=8)
            def _(k):
                sl = pl.ds(k, LANES)
                plsc.addupdate_scatter(dg, [db[sl] >> 14], dwb[sl])

        pltpu.sync_copy(dg, part_hbm.at[wid])
        plsc.subcore_barrier()
        nsl = NP // 16
        sbase = sid * nsl
        slc = pl.ds(sbase, nsl)
        pltpu.sync_copy(part_hbm.at[cid].at[slc], dg.at[slc])
        for t in range(1, 16):
            pltpu.sync_copy(part_hbm.at[2 * t + cid].at[slc], tmp)
            for r in range(nsl // LANES):
                dg[pl.ds(sbase + r * LANES, LANES)] = (
                    dg[pl.ds(sbase + r * LANES, LANES)]
                    + tmp[pl.ds(r * LANES, LANES)])
        pltpu.sync_copy(dg.at[slc], degsc_hbm.at[cid].at[slc])
        plsc.subcore_barrier()
        pltpu.sync_copy(degsc_hbm.at[cid], dg)

        for _cp in tcps:
            _cp.wait()

        @plsc.parallel_loop(0, NP, step=LANES, unroll=8)
        def _(i):
            sl = pl.ds(i, LANES)
            y = _rsqrt16(dg[sl])
            ds[sl] = y
            dg[sl] = y * y
            a = h0[sl] * y
            b_ = h1[sl] * y
            h0[sl] = a
            h1[sl] = b_
            t0[sl] = a
            t1[sl] = b_

        def hop1_chunk(p_, w_, k):
            sl = pl.ds(k, LANES)
            p = p_[sl]
            r = p & 0x3FFF
            c = p >> 14
            w = w_[sl]
            plsc.addupdate_scatter(t0, [c], w * plsc.load_gather(h0, [r]))
            plsc.addupdate_scatter(t1, [c], w * plsc.load_gather(h1, [r]))

        edge_pass(hop1_chunk)

        @plsc.parallel_loop(0, NP, step=LANES, unroll=8)
        def _(i):
            sl = pl.ds(i, LANES)
            inv = dg[sl]
            a = t0[sl] * inv
            b_ = t1[sl] * inv
            t0[sl] = a
            t1[sl] = b_
            h0[sl] = a
            h1[sl] = b_

        def hop2_chunk(p_, w_, k):
            sl = pl.ds(k, LANES)
            p = p_[sl]
            r = p & 0x3FFF
            c = p >> 14
            w = w_[sl]
            plsc.addupdate_scatter(h0, [c], w * plsc.load_gather(t0, [r]))
            plsc.addupdate_scatter(h1, [c], w * plsc.load_gather(t1, [r]))

        edge_pass(hop2_chunk)

        @plsc.parallel_loop(0, NP, step=LANES, unroll=8)
        def _(i):
            sl = pl.ds(i, LANES)
            y = ds[sl]
            h0[sl] = h0[sl] * y
            h1[sl] = h1[sl] * y

        for q in range(4):
            qs = pl.ds(2560 * q, 2560)
            pltpu.sync_copy(h0.at[qs], out_hbm.at[j0].at[qs])
            pltpu.sync_copy(h1.at[qs], out_hbm.at[j0 + 1].at[qs])

    return sc_prop(xwT, packed, ew)[0]


def _lsm_body(h_ref, b_ref, o_ref):
    z = h_ref[...] + b_ref[...]
    m = jnp.max(z, axis=0, keepdims=True)
    zs = z - m
    ls = zs - jnp.log(jnp.sum(jnp.exp(zs), axis=0, keepdims=True))
    o_ref[...] = ls.T


def _log_softmax_fm(h, b):
    bn = 2048
    return pl.pallas_call(
        _lsm_body,
        grid=(NP // bn,),
        in_specs=[
            pl.BlockSpec((F_OUT, bn), lambda i: (0, i)),
            pl.BlockSpec((F_OUT, 1), lambda i: (0, 0)),
        ],
        out_specs=pl.BlockSpec((bn, F_OUT), lambda i: (i, 0)),
        out_shape=jax.ShapeDtypeStruct((NP, F_OUT), jnp.float32),
    )(h, b)


def kernel(x, edge_index, edge_attr, W, b):
    x_pad = jnp.pad(x, ((0, NP - N), (0, 0)))
    row = edge_index[0]
    col = edge_index[1]

    packed = _pack_edges(row, col)
    xwT = _feature_major_xw(x_pad, W)
    hT = _sc_propagate(xwT, packed, edge_attr)
    out = _log_softmax_fm(hT, b.reshape(F_OUT, 1))
    return out[:N]


# speedup vs baseline: 1.0139x; 1.0139x over previous
import dataclasses
import functools

import jax
import jax.numpy as jnp
from jax import lax
from jax.experimental import pallas as pl
from jax.experimental.pallas import tpu as pltpu
from jax.experimental.pallas import tpu_sc as plsc

N = 10000
E = 320000
F_IN = 128
F_OUT = 64
NP = 10240
BLK = 6400
NBLK = E // BLK
LANES = 16


def _matmul_body(w_ref, x_ref, o_ref):
    o_ref[...] = lax.dot_general(
        w_ref[...], x_ref[...],
        (((1,), (1,)), ((), ())),
        preferred_element_type=jnp.float32,
        precision=lax.Precision.HIGHEST,
    )


def _pack_body(r_ref, c_ref, o_ref):
    o_ref[...] = r_ref[...] | (c_ref[...] << 14)


def _pack_edges(row, col):
    r2 = row.reshape(E // 128, 128)
    c2 = col.reshape(E // 128, 128)
    packed = pl.pallas_call(
        _pack_body,
        out_shape=jax.ShapeDtypeStruct((E // 128, 128), jnp.int32),
    )(r2, c2)
    return packed.reshape(E)


def _feature_major_xw(x_pad, W):
    bn = 2048
    return pl.pallas_call(
        _matmul_body,
        grid=(NP // bn,),
        in_specs=[
            pl.BlockSpec((F_OUT, F_IN), lambda i: (0, 0)),
            pl.BlockSpec((bn, F_IN), lambda i: (i, 0)),
        ],
        out_specs=pl.BlockSpec((F_OUT, bn), lambda i: (0, i)),
        out_shape=jax.ShapeDtypeStruct((F_OUT, NP), jnp.float32),
    )(W, x_pad)


def _rsqrt16(d):
    i = lax.bitcast_convert_type(d, jnp.int32)
    y = lax.bitcast_convert_type(jnp.int32(0x5F3759DF) - (i >> 1), jnp.float32)
    half = d * 0.5
    for _ in range(3):
        y = y * (1.5 - half * y * y)
    return y


def _sc_propagate(xwT, packed, ew):
    mesh = plsc.VectorSubcoreMesh(
        core_axis_name="c", subcore_axis_name="s", num_cores=2, num_subcores=16
    )
    cp = pltpu.CompilerParams()
    if "needs_layout_passes" in pltpu.CompilerParams.__dataclass_fields__:
        cp = dataclasses.replace(cp, needs_layout_passes=False)

    @functools.partial(
        pl.kernel,
        compiler_params=cp,
        out_type=(
            jax.ShapeDtypeStruct((F_OUT, NP), jnp.float32),
            jax.ShapeDtypeStruct((32, NP), jnp.float32),
            jax.ShapeDtypeStruct((2, NP), jnp.float32),
        ),
        mesh=mesh,
        scratch_types=[
            pltpu.VMEM((NP,), jnp.float32),
            pltpu.VMEM((NP,), jnp.float32),
            pltpu.VMEM((NP,), jnp.float32),
            pltpu.VMEM((NP,), jnp.float32),
            pltpu.VMEM((NP,), jnp.float32),
            pltpu.VMEM((NP,), jnp.float32),
            pltpu.VMEM((BLK,), jnp.int32),
            pltpu.VMEM((BLK,), jnp.float32),
            pltpu.VMEM((BLK,), jnp.int32),
            pltpu.VMEM((BLK,), jnp.float32),
            pltpu.VMEM((4000,), jnp.int32),
            pltpu.VMEM((4000,), jnp.float32),
            pltpu.VMEM((NP // 16,), jnp.float32),
            pltpu.SemaphoreType.DMA,
            pltpu.SemaphoreType.DMA,
            pltpu.SemaphoreType.DMA,
        ],
    )
    def sc_prop(xw_hbm, pk_hbm, ew_hbm, out_hbm, part_hbm, degsc_hbm,
                h0, h1, t0, t1, dg, ds,
                pb0, wb0, pb1, wb1, db, dwb, tmp, semA, semB, tsem):
        sid = lax.axis_index("s")
        cid = lax.axis_index("c")
        wid = sid * 2 + cid
        j0 = wid * 2

        tcps = []
        for q in range(4):
            qs = pl.ds(2560 * q, 2560)
            tcps.append(pltpu.async_copy(xw_hbm.at[j0].at[qs], h0.at[qs], tsem))
            tcps.append(
                pltpu.async_copy(xw_hbm.at[j0 + 1].at[qs], h1.at[qs], tsem))

        def issue(e0, p_, w_, sem):
            sl = pl.ds(e0, BLK)
            pltpu.async_copy(pk_hbm.at[sl], p_, sem)
            pltpu.async_copy(ew_hbm.at[sl], w_, sem)

        def drain(p_, w_, sem):
            sl = pl.ds(0, BLK)
            pltpu.make_async_copy(pk_hbm.at[sl], p_, sem).wait()
            pltpu.make_async_copy(ew_hbm.at[sl], w_, sem).wait()

        def edge_pass(chunk_body):
            issue(0, pb0, wb0, semA)

            @pl.loop(0, E, step=2 * BLK)
            def _(e0):
                issue(e0 + BLK, pb1, wb1, semB)
                drain(pb0, wb0, semA)

                @plsc.parallel_loop(0, BLK, step=LANES, unroll=8)
                def _(k):
                    chunk_body(pb0, wb0, k)

                @pl.when(e0 + 2 * BLK < E)
                def _():
                    issue(e0 + 2 * BLK, pb0, wb0, semA)

                drain(pb1, wb1, semB)

                @plsc.parallel_loop(0, BLK, step=LANES, unroll=8)
                def _(k):
                    chunk_body(pb1, wb1, k)

        init16 = jnp.full((LANES,), 1.0, jnp.float32) * (sid == 0).astype(
            jnp.float32)

        @plsc.parallel_loop(0, NP, step=LANES)
        def _(i):
            dg[pl.ds(i, LANES)] = init16

        dbase = sid * (E // 16)
        for rblk in range(5):
            off = dbase + rblk * 4000
            pltpu.async_copy(pk_hbm.at[pl.ds(off, 4000)], db, semA)
            pltpu.async_copy(ew_hbm.at[pl.ds(off, 4000)], dwb, semA)
            pltpu.make_async_copy(pk_hbm.at[pl.ds(0, 4000)], db, semA).wait()
            pltpu.make_async_copy(ew_hbm.at[pl.ds(0, 4000)], dwb, semA).wait()

            @plsc.parallel_loop(0, 4000, step=LANES, unroll=8)
            def _(k):
                sl = pl.ds(k, LANES)
                plsc.addupdate_scatter(dg, [db[sl] >> 14], dwb[sl])

        pltpu.sync_copy(dg, part_hbm.at[wid])
        plsc.subcore_barrier()
        nsl = NP // 16
        sbase = sid * nsl
        slc = pl.ds(sbase, nsl)
        pltpu.sync_copy(part_hbm.at[cid].at[slc], dg.at[slc])
        for t in range(1, 16):
            pltpu.sync_copy(part_hbm.at[2 * t + cid].at[slc], tmp)
            for r in range(nsl // LANES):
                dg[pl.ds(sbase + r * LANES, LANES)] = (
                    dg[pl.ds(sbase + r * LANES, LANES)]
                    + tmp[pl.ds(r * LANES, LANES)])
        pltpu.sync_copy(dg.at[slc], degsc_hbm.at[cid].at[slc])
        plsc.subcore_barrier()
        pltpu.sync_copy(degsc_hbm.at[cid], dg)

        for _cp in tcps:
            _cp.wait()

        @plsc.parallel_loop(0, NP, step=LANES, unroll=8)
        def _(i):
            sl = pl.ds(i, LANES)
            y = _rsqrt16(dg[sl])
            ds[sl] = y
            dg[sl] = y * y
            a = h0[sl] * y
            b_ = h1[sl] * y
            h0[sl] = a
            h1[sl] = b_
            t0[sl] = a
            t1[sl] = b_

        def hop1_chunk(p_, w_, k):
            sl = pl.ds(k, LANES)
            p = p_[sl]
            r = p & 0x3FFF
            c = p >> 14
            w = w_[sl]
            plsc.addupdate_scatter(t0, [c], w * plsc.load_gather(h0, [r]))
            plsc.addupdate_scatter(t1, [c], w * plsc.load_gather(h1, [r]))

        edge_pass(hop1_chunk)

        @plsc.parallel_loop(0, NP, step=LANES, unroll=8)
        def _(i):
            sl = pl.ds(i, LANES)
            inv = dg[sl]
            a = t0[sl] * inv
            b_ = t1[sl] * inv
            t0[sl] = a
            t1[sl] = b_
            h0[sl] = a
            h1[sl] = b_

        def hop2_chunk(p_, w_, k):
            sl = pl.ds(k, LANES)
            p = p_[sl]
            r = p & 0x3FFF
            c = p >> 14
            w = w_[sl]
            plsc.addupdate_scatter(h0, [c], w * plsc.load_gather(t0, [r]))
            plsc.addupdate_scatter(h1, [c], w * plsc.load_gather(t1, [r]))

        edge_pass(hop2_chunk)

        @plsc.parallel_loop(0, NP, step=LANES, unroll=8)
        def _(i):
            sl = pl.ds(i, LANES)
            y = ds[sl]
            h0[sl] = h0[sl] * y
            h1[sl] = h1[sl] * y

        for q in range(4):
            qs = pl.ds(2560 * q, 2560)
            pltpu.sync_copy(h0.at[qs], out_hbm.at[j0].at[qs])
            pltpu.sync_copy(h1.at[qs], out_hbm.at[j0 + 1].at[qs])

    return sc_prop(xwT, packed, ew)[0]


def _lsm_body(h_ref, b_ref, o_ref):
    z = h_ref[...] + b_ref[...]
    m = jnp.max(z, axis=0, keepdims=True)
    zs = z - m
    o_ref[...] = zs - jnp.log(jnp.sum(jnp.exp(zs), axis=0, keepdims=True))


def _log_softmax_fm(h, b):
    bn = 2048
    return pl.pallas_call(
        _lsm_body,
        grid=(NP // bn,),
        in_specs=[
            pl.BlockSpec((F_OUT, bn), lambda i: (0, i)),
            pl.BlockSpec((F_OUT, 1), lambda i: (0, 0)),
        ],
        out_specs=pl.BlockSpec((F_OUT, bn), lambda i: (0, i)),
        out_shape=jax.ShapeDtypeStruct((F_OUT, NP), jnp.float32),
    )(h, b)


def kernel(x, edge_index, edge_attr, W, b):
    x_pad = jnp.pad(x, ((0, NP - N), (0, 0)))
    row = edge_index[0]
    col = edge_index[1]

    packed = _pack_edges(row, col)
    xwT = _feature_major_xw(x_pad, W)
    hT = _sc_propagate(xwT, packed, edge_attr)
    outT = _log_softmax_fm(hT, b.reshape(F_OUT, 1))
    return outT.T[:N]
